# Initial kernel scaffold; baseline (speedup 1.0000x reference)
import jax
import jax.numpy as jnp
from jax.experimental import pallas as pl


def kernel(encoding_indices, table):
    raise NotImplementedError("write your pallas kernel here")


# speedup vs baseline: 2.9635x; 2.9635x over previous
import jax
import jax.numpy as jnp
from jax import lax
from jax.experimental import pallas as pl
from jax.experimental.pallas import tpu as pltpu
from jax.experimental.pallas import tpu_sc as plsc

NUM_CORES = 2
NUM_SUBCORES = 16
NUM_WORKERS = NUM_CORES * NUM_SUBCORES
CHUNK = 128


def _gather_body(idx_hbm, table_hbm, out_hbm, idx_v, rows_v, gsem):
    wid = lax.axis_index("s") * NUM_CORES + lax.axis_index("c")
    nch = idx_hbm.shape[1]
    b_per_w = nch * CHUNK
    pltpu.sync_copy(idx_hbm.at[wid], idx_v)

    def body(c, carry):
        pltpu.async_copy(table_hbm.at[idx_v.at[c]], rows_v, gsem).wait()
        pltpu.sync_copy(rows_v, out_hbm.at[pl.ds(wid * b_per_w + c * CHUNK, CHUNK)])
        return carry

    lax.fori_loop(0, nch, body, 0)


def kernel(encoding_indices, table):
    B, S = encoding_indices.shape
    V, D = table.shape
    n = B * S
    nch = n // (NUM_WORKERS * CHUNK)
    idx = encoding_indices.reshape(NUM_WORKERS, nch, CHUNK).astype(jnp.int32)
    mesh = plsc.VectorSubcoreMesh(core_axis_name="c", subcore_axis_name="s")
    out = pl.kernel(
        _gather_body,
        out_type=jax.ShapeDtypeStruct((n, D), jnp.float32),
        mesh=mesh,
        scratch_types=[
            pltpu.VMEM((nch, CHUNK), jnp.int32),
            pltpu.VMEM((CHUNK, D), jnp.float32),
            pltpu.SemaphoreType.DMA,
        ],
    )(idx, table)
    return out.reshape(B, S, D)


# speedup vs baseline: 3.2295x; 1.0898x over previous
import jax
import jax.numpy as jnp
from jax import lax
from jax.experimental import pallas as pl
from jax.experimental.pallas import tpu as pltpu
from jax.experimental.pallas import tpu_sc as plsc

NUM_CORES = 2
NUM_SUBCORES = 16
NUM_WORKERS = NUM_CORES * NUM_SUBCORES
CHUNK = 128


def _gather_body(idx_hbm, table_hbm, out_hbm, idx_v, rows0, rows1, g0, g1, s0, s1):
    wid = lax.axis_index("s") * NUM_CORES + lax.axis_index("c")
    nch = idx_hbm.shape[1]
    base = wid * (nch * CHUNK)
    nrounds = nch // 2

    pltpu.sync_copy(idx_hbm.at[wid], idx_v)

    def gather_start(buf, sem, c):
        pltpu.async_copy(table_hbm.at[idx_v.at[c]], buf, sem)

    def gather_wait(buf, sem, c):
        pltpu.make_async_copy(table_hbm.at[idx_v.at[c]], buf, sem).wait()

    def out_slice(c):
        return out_hbm.at[pl.ds(base + c * CHUNK, CHUNK)]

    def store_start(buf, sem, c):
        pltpu.async_copy(buf, out_slice(c), sem)

    def store_wait(buf, sem, c):
        pltpu.make_async_copy(buf, out_slice(c), sem).wait()

    gather_start(rows0, g0, 0)
    gather_start(rows1, g1, 1)

    def round_body(r, carry):
        c0 = 2 * r
        c1 = c0 + 1
        gather_wait(rows0, g0, c0)
        store_start(rows0, s0, c0)
        gather_wait(rows1, g1, c1)
        store_start(rows1, s1, c1)

        @pl.when(r < nrounds - 1)
        def _prefetch():
            store_wait(rows0, s0, c0)
            gather_start(rows0, g0, c0 + 2)
            store_wait(rows1, s1, c1)
            gather_start(rows1, g1, c1 + 2)

        return carry

    lax.fori_loop(0, nrounds, round_body, 0)
    store_wait(rows0, s0, nch - 2)
    store_wait(rows1, s1, nch - 1)


def kernel(encoding_indices, table):
    B, S = encoding_indices.shape
    V, D = table.shape
    n = B * S
    nch = n // (NUM_WORKERS * CHUNK)
    idx = encoding_indices.reshape(NUM_WORKERS, nch, CHUNK).astype(jnp.int32)
    mesh = plsc.VectorSubcoreMesh(core_axis_name="c", subcore_axis_name="s")
    out = pl.kernel(
        _gather_body,
        out_type=jax.ShapeDtypeStruct((n, D), jnp.float32),
        mesh=mesh,
        scratch_types=[
            pltpu.VMEM((nch, CHUNK), jnp.int32),
            pltpu.VMEM((CHUNK, D), jnp.float32),
            pltpu.VMEM((CHUNK, D), jnp.float32),
            pltpu.SemaphoreType.DMA,
            pltpu.SemaphoreType.DMA,
            pltpu.SemaphoreType.DMA,
            pltpu.SemaphoreType.DMA,
        ],
    )(idx, table)
    return out.reshape(B, S, D)


# speedup vs baseline: 3.2962x; 1.0207x over previous
import jax
import jax.numpy as jnp
from jax import lax
from jax.experimental import pallas as pl
from jax.experimental.pallas import tpu as pltpu
from jax.experimental.pallas import tpu_sc as plsc

NUM_CORES = 2
NUM_SUBCORES = 16
NUM_WORKERS = NUM_CORES * NUM_SUBCORES
CHUNK = 128
NBUF = 5


def _gather_body(idx_hbm, table_hbm, out_hbm, idx_v, *scratch):
    bufs = scratch[:NBUF]
    gsems = scratch[NBUF : 2 * NBUF]
    ssems = scratch[2 * NBUF :]
    wid = lax.axis_index("s") * NUM_CORES + lax.axis_index("c")
    nch = idx_hbm.shape[1]
    base = wid * (nch * CHUNK)
    nrounds = nch // NBUF

    pltpu.sync_copy(idx_hbm.at[wid], idx_v)

    def gather_start(b, c):
        pltpu.async_copy(table_hbm.at[idx_v.at[c]], bufs[b], gsems[b])

    def gather_wait(b, c):
        pltpu.make_async_copy(table_hbm.at[idx_v.at[c]], bufs[b], gsems[b]).wait()

    def out_slice(c):
        return out_hbm.at[pl.ds(base + c * CHUNK, CHUNK)]

    def store_start(b, c):
        pltpu.async_copy(bufs[b], out_slice(c), ssems[b])

    def store_wait(b, c):
        pltpu.make_async_copy(bufs[b], out_slice(c), ssems[b]).wait()

    for b in range(NBUF):
        gather_start(b, b)

    def round_body(r, carry):
        cbase = NBUF * r
        for b in range(NBUF):
            gather_wait(b, cbase + b)
            store_start(b, cbase + b)

        @pl.when(r < nrounds - 1)
        def _prefetch():
            for b in range(NBUF):
                store_wait(b, cbase + b)
                gather_start(b, cbase + NBUF + b)

        return carry

    lax.fori_loop(0, nrounds, round_body, 0)
    for b in range(NBUF):
        store_wait(b, nch - NBUF + b)


def kernel(encoding_indices, table):
    B, S = encoding_indices.shape
    V, D = table.shape
    n = B * S
    nch = n // (NUM_WORKERS * CHUNK)
    idx = encoding_indices.reshape(NUM_WORKERS, nch, CHUNK).astype(jnp.int32)
    mesh = plsc.VectorSubcoreMesh(core_axis_name="c", subcore_axis_name="s")
    out = pl.kernel(
        _gather_body,
        out_type=jax.ShapeDtypeStruct((n, D), jnp.float32),
        mesh=mesh,
        scratch_types=(
            [pltpu.VMEM((nch, CHUNK), jnp.int32)]
            + [pltpu.VMEM((CHUNK, D), jnp.float32) for _ in range(NBUF)]
            + [pltpu.SemaphoreType.DMA for _ in range(2 * NBUF)]
        ),
    )(idx, table)
    return out.reshape(B, S, D)


# speedup vs baseline: 5.8918x; 1.7874x over previous
import jax
import jax.numpy as jnp
from jax import lax
from jax.experimental import pallas as pl
from jax.experimental.pallas import tpu as pltpu
from jax.experimental.pallas import tpu_sc as plsc

NUM_CORES = 2
NUM_SUBCORES = 16
NUM_WORKERS = NUM_CORES * NUM_SUBCORES
CB = 4
NBUF = 4


def _gather_body(idx_hbm, table_hbm, out_hbm, idx_v, *scratch):
    bufs = scratch[:NBUF]
    gsems = scratch[NBUF : 2 * NBUF]
    ssems = scratch[2 * NBUF :]
    wid = lax.axis_index("s") * NUM_CORES + lax.axis_index("c")
    rows_per_w = idx_hbm.shape[1]
    base = wid * rows_per_w
    nchunks = rows_per_w // CB
    nrounds = nchunks // NBUF

    pltpu.sync_copy(idx_hbm.at[wid], idx_v)

    def gathers_start(b, c):
        for j in range(CB):
            pltpu.async_copy(table_hbm.at[idx_v.at[c * CB + j]], bufs[b].at[j], gsems[b])

    def gathers_wait(b, c):
        for j in range(CB):
            pltpu.make_async_copy(
                table_hbm.at[idx_v.at[c * CB + j]], bufs[b].at[j], gsems[b]
            ).wait()

    def store_start(b, c):
        pltpu.async_copy(bufs[b], out_hbm.at[pl.ds(base + c * CB, CB)], ssems[b])

    def store_wait(b, c):
        pltpu.make_async_copy(bufs[b], out_hbm.at[pl.ds(base + c * CB, CB)], ssems[b]).wait()

    for b in range(NBUF):
        gathers_start(b, b)

    def round_body(r, carry):
        cbase = NBUF * r
        for b in range(NBUF):
            gathers_wait(b, cbase + b)
            store_start(b, cbase + b)

        @pl.when(r < nrounds - 1)
        def _prefetch():
            for b in range(NBUF):
                store_wait(b, cbase + b)
                gathers_start(b, cbase + NBUF + b)

        return carry

    lax.fori_loop(0, nrounds, round_body, 0)
    for b in range(NBUF):
        store_wait(b, nchunks - NBUF + b)


def kernel(encoding_indices, table):
    B, S = encoding_indices.shape
    V, D = table.shape
    rows_per_w = B // NUM_WORKERS
    idx = encoding_indices.reshape(NUM_WORKERS, rows_per_w, S).astype(jnp.int32)
    mesh = plsc.VectorSubcoreMesh(core_axis_name="c", subcore_axis_name="s")
    out = pl.kernel(
        _gather_body,
        out_type=jax.ShapeDtypeStruct((B, S, D), jnp.float32),
        mesh=mesh,
        scratch_types=(
            [pltpu.VMEM((rows_per_w, S), jnp.int32)]
            + [pltpu.VMEM((CB, S, D), jnp.float32) for _ in range(NBUF)]
            + [pltpu.SemaphoreType.DMA for _ in range(2 * NBUF)]
        ),
    )(idx, table)
    return out
